# Initial kernel scaffold; baseline (speedup 1.0000x reference)
import jax
import jax.numpy as jnp
from jax.experimental import pallas as pl


def kernel(x, Wq, keys, expert_down, expert_up, norm_weight):
    raise NotImplementedError("write your pallas kernel here")


# speedup vs baseline: 5.3343x; 5.3343x over previous
import functools

import jax
import jax.numpy as jnp
from jax import lax
from jax.experimental import pallas as pl
from jax.experimental.pallas import tpu as pltpu
from jax.experimental.pallas import tpu_sc as plsc

B, T, D = 1, 2048, 1024
H = 8
DK = 32
NUM_KEYS = 256
K = 8
EPS = 1e-6
TB = 256
HK = H * K


def _topk8(v, n):
    iota = lax.broadcasted_iota(jnp.int32, v.shape, 1)
    scs, ixs = [], []
    for _ in range(K):
        m = jnp.max(v, axis=1, keepdims=True)
        cand = jnp.where(v >= m, iota, n)
        a = jnp.min(cand, axis=1, keepdims=True)
        scs.append(m)
        ixs.append(a)
        v = jnp.where(iota == a, -jnp.inf, v)
    return jnp.concatenate(scs, axis=1), jnp.concatenate(ixs, axis=1)


def _tc_body(x_ref, w1t_ref, w2t_ref, k1t_ref, k2t_ref, nw_ref,
             xn_ref, fidx_ref, fsc_ref):
    xb = x_ref[...]
    ms = jnp.mean(xb * xb, axis=-1, keepdims=True)
    xn = xb * lax.rsqrt(ms + EPS) * nw_ref[...]
    xn_ref[...] = xn
    for h in range(H):
        q1 = jnp.dot(xn, w1t_ref[h], preferred_element_type=jnp.float32)
        q2 = jnp.dot(xn, w2t_ref[h], preferred_element_type=jnp.float32)
        s1 = jnp.dot(q1, k1t_ref[h], preferred_element_type=jnp.float32)
        s2 = jnp.dot(q2, k2t_ref[h], preferred_element_type=jnp.float32)
        sc1, ix1 = _topk8(s1, NUM_KEYS)
        sc2, ix2 = _topk8(s2, NUM_KEYS)
        allsc = jnp.concatenate(
            [sc1[:, i:i + 1] + sc2 for i in range(K)], axis=1)
        allix = jnp.concatenate(
            [ix1[:, i:i + 1] * NUM_KEYS + ix2 for i in range(K)], axis=1)
        iota = lax.broadcasted_iota(jnp.int32, allsc.shape, 1)
        fss, fis = [], []
        v = allsc
        for _ in range(K):
            m = jnp.max(v, axis=1, keepdims=True)
            cand = jnp.where(v >= m, iota, K * K)
            a = jnp.min(cand, axis=1, keepdims=True)
            fss.append(m)
            fis.append(jnp.sum(jnp.where(iota == a, allix, 0), axis=1,
                               keepdims=True))
            v = jnp.where(iota == a, -jnp.inf, v)
        fsc_ref[:, h * K:(h + 1) * K] = jnp.maximum(
            jnp.concatenate(fss, axis=1), 0.0)
        fidx_ref[:, h * K:(h + 1) * K] = jnp.concatenate(fis, axis=1)


def _tc_frontend(x2d, w1t, w2t, k1t, k2t, nw, interpret=False):
    grid = (T // TB,)
    return pl.pallas_call(
        _tc_body,
        grid=grid,
        in_specs=[
            pl.BlockSpec((TB, D), lambda i: (i, 0)),
            pl.BlockSpec((H, D, DK), lambda i: (0, 0, 0)),
            pl.BlockSpec((H, D, DK), lambda i: (0, 0, 0)),
            pl.BlockSpec((H, DK, NUM_KEYS), lambda i: (0, 0, 0)),
            pl.BlockSpec((H, DK, NUM_KEYS), lambda i: (0, 0, 0)),
            pl.BlockSpec((1, D), lambda i: (0, 0)),
        ],
        out_specs=[
            pl.BlockSpec((TB, D), lambda i: (i, 0)),
            pl.BlockSpec((TB, HK), lambda i: (i, 0)),
            pl.BlockSpec((TB, HK), lambda i: (i, 0)),
        ],
        out_shape=[
            jax.ShapeDtypeStruct((T, D), jnp.float32),
            jax.ShapeDtypeStruct((T, HK), jnp.int32),
            jax.ShapeDtypeStruct((T, HK), jnp.float32),
        ],
        interpret=interpret,
    )(x2d, w1t, w2t, k1t, k2t, nw)


_ERF_P = 0.3275911
_ERF_A = (0.254829592, -0.284496736, 1.421413741, -1.453152027, 1.061405429)
_INV_SQRT2 = 0.7071067811865476


def _gelu16(x):
    z = jnp.abs(x) * _INV_SQRT2
    t = 1.0 / (1.0 + _ERF_P * z)
    poly = _ERF_A[4]
    for a in (_ERF_A[3], _ERF_A[2], _ERF_A[1], _ERF_A[0]):
        poly = poly * t + a
    poly = poly * t
    erf_abs = 1.0 - poly * jnp.exp(-(z * z))
    erf = jnp.where(x < 0, -erf_abs, erf_abs)
    return 0.5 * x * (1.0 + erf)


NC = 2
NS = 16


def _sc_body(nw_tok, xn_hbm, fidx_hbm, fsc_hbm, down_hbm, up_hbm, out_hbm,
             x_v, idx_v, sc_v, rows_v, out_v, hid_v, sem_d, sem_u):
    wid = lax.axis_index("s") * NC + lax.axis_index("c")
    lanes = lax.iota(jnp.int32, 16)

    def token_body(i, _):
        t = wid * nw_tok + i
        pltpu.sync_copy(xn_hbm.at[t], x_v)
        pltpu.sync_copy(fidx_hbm.at[t], idx_v)
        pltpu.sync_copy(fsc_hbm.at[t], sc_v)
        pltpu.async_copy(down_hbm.at[idx_v], rows_v, sem_d).wait()

        for g in range(4):
            def chunk_body(c, accs):
                xc = x_v[pl.ds(c * 16, 16)]
                new = []
                for j in range(16):
                    rc = rows_v[g * 16 + j, pl.ds(c * 16, 16)]
                    new.append(accs[j] + xc * rc)
                return tuple(new)

            accs = lax.fori_loop(0, D // 16, chunk_body,
                                 tuple(jnp.zeros((16,), jnp.float32)
                                       for _ in range(16)))
            tot = jnp.zeros((16,), jnp.float32)
            for j in range(16):
                tot = jnp.where(lanes == j, jnp.sum(accs[j]), tot)
            hid = _gelu16(tot) * sc_v[pl.ds(g * 16, 16)]
            hid_v[pl.ds(g * 16, 16)] = hid

        pltpu.async_copy(up_hbm.at[idx_v], rows_v, sem_u).wait()
        for g in range(4):
            hid16 = hid_v[pl.ds(g * 16, 16)]
            hvecs = [jnp.full((16,), hid16[j], jnp.float32)
                     for j in range(16)]

            def out_chunk(c, _, g=g, hvecs=hvecs):
                oc = (out_v[pl.ds(c * 16, 16)] if g else
                      jnp.zeros((16,), jnp.float32))
                for j in range(16):
                    oc = oc + hvecs[j] * rows_v[g * 16 + j, pl.ds(c * 16, 16)]
                out_v[pl.ds(c * 16, 16)] = oc
                return 0

            lax.fori_loop(0, D // 16, out_chunk, 0)
        pltpu.sync_copy(out_v, out_hbm.at[t])
        return 0

    lax.fori_loop(0, nw_tok, token_body, 0)


def _sc_backend(xn, fidx, fsc, down, up):
    nw_tok = T // (NC * NS)
    mesh = plsc.VectorSubcoreMesh(core_axis_name="c", subcore_axis_name="s",
                                  num_cores=NC, num_subcores=NS)
    kern = pl.kernel(
        functools.partial(_sc_body, nw_tok),
        out_type=jax.ShapeDtypeStruct((T, D), jnp.float32),
        mesh=mesh,
        scratch_types=[
            pltpu.VMEM((D,), jnp.float32),
            pltpu.VMEM((HK,), jnp.int32),
            pltpu.VMEM((HK,), jnp.float32),
            pltpu.VMEM((HK, D), jnp.float32),
            pltpu.VMEM((D,), jnp.float32),
            pltpu.VMEM((HK,), jnp.float32),
            pltpu.SemaphoreType.DMA,
            pltpu.SemaphoreType.DMA,
        ],
        compiler_params=pltpu.CompilerParams(needs_layout_passes=False),
    )
    return kern(xn, fidx, fsc, down, up)


def kernel(x, Wq, keys, expert_down, expert_up, norm_weight):
    x2d = x.reshape(T, D)
    wr = Wq.reshape(H, 2, DK, D)
    w1t = wr[:, 0].transpose(0, 2, 1)
    w2t = wr[:, 1].transpose(0, 2, 1)
    k1t = keys[:, :, 0, :].transpose(0, 2, 1)
    k2t = keys[:, :, 1, :].transpose(0, 2, 1)
    xn, fidx, fsc = _tc_frontend(x2d, w1t, w2t, k1t, k2t,
                                 norm_weight.reshape(1, D))
    out = _sc_backend(xn, fidx, fsc, expert_down, expert_up)
    return out.reshape(B, T, D)
